# baseline (speedup 1.0000x reference)
import functools

import jax
import jax.numpy as jnp
from jax import lax
from jax.experimental import pallas as pl
from jax.experimental.pallas import tpu as pltpu
from jax.experimental.pallas import tpu_sc as plsc

_TOTAL_NODES = 16384
_TOTAL_EDGES = 524288
_N = 10000
_E = 320000
_D = 128

_NW = 32

_X_WORDS = _N * _D
_XP_WORDS = _TOTAL_NODES * _D
_ZERO_WORDS = _XP_WORDS - _X_WORDS

_XC = _X_WORDS // _NW
_ZC = _ZERO_WORDS // _NW
_ZD = _ZC // 4

_EC = _E // _NW
_FC = (_TOTAL_EDGES - _E) // _NW

_NM_WORDS = _TOTAL_NODES // 4
_EM_WORDS = _TOTAL_EDGES // 4
_NM_C = _NM_WORDS // _NW
_EM_C = _EM_WORDS // _NW
_NM_B = _N // 4
_EM_B = _E // 4
_ONES4 = 0x01010101

_mesh = plsc.VectorSubcoreMesh(core_axis_name="c", subcore_axis_name="s")


@functools.partial(
    pl.kernel,
    mesh=_mesh,
    out_type=[
        jax.ShapeDtypeStruct((_XP_WORDS,), jnp.float32),
        jax.ShapeDtypeStruct((2 * _TOTAL_EDGES,), jnp.int32),
        jax.ShapeDtypeStruct((_NM_WORDS,), jnp.int32),
        jax.ShapeDtypeStruct((_EM_WORDS,), jnp.int32),
    ],
    scratch_types=[
        pltpu.VMEM((_XC,), jnp.float32),
        pltpu.VMEM((2 * _EC,), jnp.int32),
        pltpu.VMEM((_ZD,), jnp.float32),
        pltpu.VMEM((_FC,), jnp.int32),
        pltpu.VMEM((_EM_C,), jnp.int32),
        pltpu.VMEM((_NM_C,), jnp.int32),
        pltpu.SemaphoreType.DMA,
        pltpu.SemaphoreType.DMA,
        pltpu.SemaphoreType.DMA,
    ],
)
def _pad_sc(x_hbm, e_hbm, xp_hbm, ep_hbm, nm_hbm, em_hbm,
            xbuf, ebuf, zbuf, fbuf, embuf, nmbuf, sem_x, sem_e, sem_o):
    wid = lax.axis_index("s") * 2 + lax.axis_index("c")

    cp_x = pltpu.async_copy(x_hbm.at[pl.ds(wid * _XC, _XC)], xbuf, sem_x)
    cp_e0 = pltpu.async_copy(
        e_hbm.at[pl.ds(wid * _EC, _EC)], ebuf.at[pl.ds(0, _EC)], sem_e)
    cp_e1 = pltpu.async_copy(
        e_hbm.at[pl.ds(_E + wid * _EC, _EC)], ebuf.at[pl.ds(_EC, _EC)], sem_e)

    zero16 = jnp.zeros((16,), jnp.float32)
    n16 = jnp.full((16,), _N, jnp.int32)
    one16 = jnp.full((16,), _ONES4, jnp.int32)
    z16 = jnp.zeros((16,), jnp.int32)
    lane = jnp.arange(16, dtype=jnp.int32)

    def fill_zero(i, c):
        zbuf[pl.ds(i * 16, 16)] = zero16
        return c
    lax.fori_loop(0, _ZD // 16, fill_zero, 0)

    def fill_n(i, c):
        fbuf[pl.ds(i * 16, 16)] = n16
        return c
    lax.fori_loop(0, _FC // 16, fill_n, 0)

    em_base = wid * _EM_C

    def fill_em(i, c):
        idx = em_base + i * 16 + lane
        embuf[pl.ds(i * 16, 16)] = jnp.where(idx < _EM_B, one16, z16)
        return c
    lax.fori_loop(0, _EM_C // 16, fill_em, 0)

    nm_base = wid * _NM_C

    def fill_nm(i, c):
        idx = nm_base + i * 16 + lane
        nmbuf[pl.ds(i * 16, 16)] = jnp.where(idx < _NM_B, one16, z16)
        return c
    lax.fori_loop(0, _NM_C // 16, fill_nm, 0)

    outs = []
    for k in range(4):
        dst = pl.ds(_X_WORDS + wid * _ZC + k * _ZD, _ZD)
        outs.append(pltpu.async_copy(zbuf, xp_hbm.at[dst], sem_o))
    outs.append(pltpu.async_copy(
        fbuf, ep_hbm.at[pl.ds(_E + wid * _FC, _FC)], sem_o))
    outs.append(pltpu.async_copy(
        fbuf, ep_hbm.at[pl.ds(_TOTAL_EDGES + _E + wid * _FC, _FC)], sem_o))
    outs.append(pltpu.async_copy(
        embuf, em_hbm.at[pl.ds(em_base, _EM_C)], sem_o))
    outs.append(pltpu.async_copy(
        nmbuf, nm_hbm.at[pl.ds(nm_base, _NM_C)], sem_o))

    cp_x.wait()
    outs.append(pltpu.async_copy(
        xbuf, xp_hbm.at[pl.ds(wid * _XC, _XC)], sem_o))
    cp_e0.wait()
    cp_e1.wait()
    outs.append(pltpu.async_copy(
        ebuf.at[pl.ds(0, _EC)], ep_hbm.at[pl.ds(wid * _EC, _EC)], sem_o))
    outs.append(pltpu.async_copy(
        ebuf.at[pl.ds(_EC, _EC)],
        ep_hbm.at[pl.ds(_TOTAL_EDGES + wid * _EC, _EC)], sem_o))

    for cp in outs:
        cp.wait()


def kernel(x, edge_index):
    ei = edge_index.astype(jnp.int32)
    xp_flat, ep_flat, nm32, em32 = _pad_sc(x.reshape(-1), ei.reshape(-1))

    x_padded = xp_flat.reshape(_TOTAL_NODES, _D)
    edge_index_padded = ep_flat.reshape(2, _TOTAL_EDGES).astype(edge_index.dtype)
    node_mask = lax.bitcast_convert_type(
        nm32, jnp.uint8).reshape(_TOTAL_NODES).astype(jnp.bool_)
    edge_mask = lax.bitcast_convert_type(
        em32, jnp.uint8).reshape(_TOTAL_EDGES).astype(jnp.bool_)
    node_sizes = jnp.array([_N, _TOTAL_NODES - _N], dtype=jnp.int64)
    edge_sizes = jnp.array([_E, _TOTAL_EDGES - _E], dtype=jnp.int64)

    return (x_padded, edge_index_padded, node_mask, edge_mask,
            node_sizes, edge_sizes)


# speedup vs baseline: 4.6161x; 4.6161x over previous
import functools

import jax
import jax.numpy as jnp
from jax import lax
from jax.experimental import pallas as pl
from jax.experimental.pallas import tpu as pltpu
from jax.experimental.pallas import tpu_sc as plsc

_TOTAL_NODES = 16384
_TOTAL_EDGES = 524288
_N = 10000
_E = 320000
_D = 128

_NW = 32
_CW = 25
_CR = _N // _CW
_CCH = 2
_CCR = _CR // _CCH
_ZW = _NW - _CW
_ZR = (_TOTAL_NODES - _N) // _ZW
_ZCH = 6
_ZCR = _ZR // _ZCH

_mesh = plsc.VectorSubcoreMesh(core_axis_name="c", subcore_axis_name="s")


@functools.partial(
    pl.kernel,
    mesh=_mesh,
    out_type=jax.ShapeDtypeStruct((_TOTAL_NODES, _D), jnp.float32),
    scratch_types=[
        pltpu.VMEM((_CR, _D), jnp.float32),
        pltpu.VMEM((_ZCR, _D), jnp.float32),
        pltpu.SemaphoreType.DMA,
        pltpu.SemaphoreType.DMA,
        pltpu.SemaphoreType.DMA,
    ],
)
def _pad_x_sc(x_hbm, xp_hbm, xbuf, zbuf, s0, s1, so):
    wid = lax.axis_index("s") * 2 + lax.axis_index("c")
    sems = [s0, s1]

    @pl.when(wid < _CW)
    def _copy():
        base = wid * _CR
        ins = [
            pltpu.async_copy(
                x_hbm.at[pl.ds(base + k * _CCR, _CCR)],
                xbuf.at[pl.ds(k * _CCR, _CCR)], sems[k])
            for k in range(_CCH)
        ]
        outs = []
        for k in range(_CCH):
            ins[k].wait()
            outs.append(pltpu.async_copy(
                xbuf.at[pl.ds(k * _CCR, _CCR)],
                xp_hbm.at[pl.ds(base + k * _CCR, _CCR)], so))
        for cp in outs:
            cp.wait()

    @pl.when(wid >= _CW)
    def _zero():
        zero16 = jnp.zeros((16,), jnp.float32)

        def fill_row(r, c):
            for j in range(_D // 16):
                zbuf[r, pl.ds(j * 16, 16)] = zero16
            return c
        lax.fori_loop(0, _ZCR, fill_row, 0)

        r0 = _N + (wid - _CW) * _ZR
        outs = [
            pltpu.async_copy(
                zbuf, xp_hbm.at[pl.ds(r0 + k * _ZCR, _ZCR)], so)
            for k in range(_ZCH)
        ]
        for cp in outs:
            cp.wait()


def _edge_tc(e_ref, ep_ref, nm_ref, em_ref, ns_ref, es_ref):
    ep_ref[:, :_E] = e_ref[...]
    ep_ref[:, _E:] = jnp.full((2, _TOTAL_EDGES - _E), _N, jnp.int32)
    nm_ref[...] = lax.broadcasted_iota(jnp.int32, (_TOTAL_NODES,), 0) < _N
    em_ref[...] = lax.broadcasted_iota(jnp.int32, (_TOTAL_EDGES,), 0) < _E
    ns_ref[0] = _N
    ns_ref[1] = _TOTAL_NODES - _N
    es_ref[0] = _E
    es_ref[1] = _TOTAL_EDGES - _E


_edge_call = pl.pallas_call(
    _edge_tc,
    out_shape=(
        jax.ShapeDtypeStruct((2, _TOTAL_EDGES), jnp.int32),
        jax.ShapeDtypeStruct((_TOTAL_NODES,), jnp.bool_),
        jax.ShapeDtypeStruct((_TOTAL_EDGES,), jnp.bool_),
        jax.ShapeDtypeStruct((2,), jnp.int32),
        jax.ShapeDtypeStruct((2,), jnp.int32),
    ),
    out_specs=(
        pl.BlockSpec(memory_space=pltpu.VMEM),
        pl.BlockSpec(memory_space=pltpu.VMEM),
        pl.BlockSpec(memory_space=pltpu.VMEM),
        pl.BlockSpec(memory_space=pltpu.SMEM),
        pl.BlockSpec(memory_space=pltpu.SMEM),
    ),
    in_specs=[pl.BlockSpec(memory_space=pltpu.VMEM)],
)


def kernel(x, edge_index):
    ei = edge_index.astype(jnp.int32)
    x_padded = _pad_x_sc(x)
    ep, node_mask, edge_mask, node_sizes, edge_sizes = _edge_call(ei)
    edge_index_padded = ep.astype(edge_index.dtype)
    return (x_padded, edge_index_padded, node_mask, edge_mask,
            node_sizes, edge_sizes)
